# baseline (speedup 1.0000x reference)
import functools

import jax
import jax.numpy as jnp
from jax import lax
from jax.experimental import pallas as pl
from jax.experimental.pallas import tpu as pltpu
from jax.experimental.pallas import tpu_sc as plsc

USER = 6000
ITEM = 4000
NN = USER + ITEM
EDGES = 320000
D = 128
MODAL_ADJ_WEIGHT = 0.4
RESIDUAL_WEIGHT = 0.5

CH = 112
KB = 8
ZR = 208
NC = 2
NS = 16
NW = NC * NS
ROWS_PER_TILE = 624


def _l2n(x):
    n = jnp.sqrt(jnp.sum(x * x, axis=-1, keepdims=True))
    return x / jnp.maximum(n, 1e-12)


def _feats_body(img_ref, wimg_ref, bimg_ref, txt_ref, wtxt_ref, btxt_ref,
                u_ref, i_ref, mw_ref, vadj_ref, vimg_ref, vtxt_ref,
                t_ref, vals_ref):
    img_f = jnp.dot(img_ref[...], wimg_ref[...],
                    preferred_element_type=jnp.float32) + bimg_ref[...]
    txt_f = jnp.dot(txt_ref[...], wtxt_ref[...],
                    preferred_element_type=jnp.float32) + btxt_ref[...]
    img_n = _l2n(img_f)
    txt_n = _l2n(txt_f)
    u = u_ref[...]
    mw = mw_ref[...]
    m = jnp.exp(mw - jnp.max(mw))
    w = m / jnp.sum(m)
    w0 = w[0, 0]
    w1 = w[0, 1]
    t_ref[0:USER] = u
    t_ref[USER:NN] = i_ref[...]
    t_ref[NN:NN + USER] = u
    t_ref[NN + USER:2 * NN] = img_n
    t_ref[2 * NN:2 * NN + USER] = u
    t_ref[2 * NN + USER:3 * NN] = txt_n
    nb = EDGES // D
    vals_ref[0:nb] = vadj_ref[...]
    vals_ref[nb:2 * nb] = vimg_ref[...] * (MODAL_ADJ_WEIGHT * w0)
    vals_ref[2 * nb:3 * nb] = vtxt_ref[...] * (MODAL_ADJ_WEIGHT * w1)


_feats = pl.pallas_call(
    _feats_body,
    out_shape=(
        jax.ShapeDtypeStruct((3 * NN, D), jnp.float32),
        jax.ShapeDtypeStruct((3 * EDGES // D, D), jnp.float32),
    ),
)


def _sum2_body(p_ref, o_ref):
    o_ref[...] = p_ref[0] + p_ref[1]


_sum2 = pl.pallas_call(
    _sum2_body,
    out_shape=jax.ShapeDtypeStruct((NN, D), jnp.float32),
)


def _final_body(modal_ref, q_ref, o_ref):
    o_ref[...] = (1.0 + RESIDUAL_WEIGHT) * modal_ref[...] + q_ref[0] + q_ref[1]


_final = pl.pallas_call(
    _final_body,
    out_shape=jax.ShapeDtypeStruct((NN, D), jnp.float32),
)


def _padded_len(n_edges):
    blk = KB * NW * CH
    data = -(-n_edges // blk) * blk
    return data, data


@functools.cache
def _make_spmm(n_edges):
    data_len, _ = _padded_len(n_edges)
    cpw = data_len // (NW * CH)
    bodies = cpw // KB
    mesh = plsc.VectorSubcoreMesh(core_axis_name="c", subcore_axis_name="s")

    @functools.partial(
        pl.kernel,
        out_type=jax.ShapeDtypeStruct((NC, NN, D), jnp.float32),
        mesh=mesh,
        compiler_params=pltpu.CompilerParams(needs_layout_passes=False),
        scratch_types=[
            pltpu.VMEM((KB * CH,), jnp.int32),
            pltpu.VMEM((KB * CH,), jnp.int32),
            pltpu.VMEM((KB * CH,), jnp.float32),
            pltpu.VMEM((CH, D), jnp.float32),
            pltpu.VMEM((CH, D), jnp.float32),
            pltpu.VMEM((CH, D), jnp.float32),
            pltpu.VMEM_SHARED((NN, D), jnp.float32),
            pltpu.SemaphoreType.DMA,
            pltpu.SemaphoreType.DMA,
            pltpu.SemaphoreType.DMA,
        ],
    )
    def spmm(row_hbm, col_hbm, val_hbm, x_hbm, out_hbm,
             colv, rowv, valv, rb_0, rb_1, st32, acc, isem, gsem, ssem):
        cid = lax.axis_index("c")
        sid = lax.axis_index("s")
        wid = sid * NC + cid
        rbs = (rb_0, rb_1)

        def zero_st(i, carry):
            for g in range(D // 16):
                st32[i, pl.ds(g * 16, 16)] = jnp.zeros((16,), jnp.float32)
            return carry

        lax.fori_loop(0, CH, zero_st, 0)
        r0 = sid * ROWS_PER_TILE
        for k in range(ROWS_PER_TILE // CH):
            pltpu.sync_copy(st32, acc.at[pl.ds(r0 + k * CH, CH)])
        rem = ROWS_PER_TILE % CH
        pltpu.sync_copy(st32.at[pl.ds(0, rem)],
                        acc.at[pl.ds(r0 + ROWS_PER_TILE - rem, rem)])

        @pl.when(sid == NS - 1)
        def _zero_tail():
            pltpu.sync_copy(st32.at[pl.ds(0, NN - NS * ROWS_PER_TILE)],
                            acc.at[pl.ds(NS * ROWS_PER_TILE,
                                         NN - NS * ROWS_PER_TILE)])

        plsc.subcore_barrier()

        def scale(rbi, off):
            @plsc.parallel_loop(0, CH // 16)
            def _(i):
                vv = valv[pl.ds(off + i * 16, 16)]
                for j in range(16):
                    vj = jnp.broadcast_to(vv[j], (16,))
                    e = i * 16 + j
                    for g in range(D // 16):
                        sl = pl.ds(g * 16, 16)
                        st32[e, sl] = rbi[e, sl] * vj

        span0 = wid * cpw * CH

        def body(t, carry):
            base = span0 + t * (KB * CH)
            ic = pltpu.async_copy(col_hbm.at[pl.ds(base, KB * CH)], colv, isem)
            ir = pltpu.async_copy(row_hbm.at[pl.ds(base, KB * CH)], rowv, isem)
            iv = pltpu.async_copy(val_hbm.at[pl.ds(base, KB * CH)], valv, isem)
            HC = CH // 2

            def gather2(q):
                return (
                    pltpu.async_copy(
                        x_hbm.at[colv.at[pl.ds(q * CH, HC)]],
                        rbs[q % 2].at[pl.ds(0, HC)], gsem),
                    pltpu.async_copy(
                        x_hbm.at[colv.at[pl.ds(q * CH + HC, HC)]],
                        rbs[q % 2].at[pl.ds(HC, HC)], gsem),
                )

            ic.wait()
            g = [gather2(0), gather2(1)]
            ir.wait()
            iv.wait()
            s_prev = None
            for q in range(KB):
                g[q][0].wait()
                g[q][1].wait()
                if s_prev is not None:
                    s_prev.wait()
                scale(rbs[q % 2], q * CH)
                if q + 2 < KB:
                    g.append(gather2(q + 2))
                s_prev = pltpu.async_copy(
                    st32, acc.at[rowv.at[pl.ds(q * CH, CH)]], ssem, add=True)
            s_prev.wait()
            return carry

        lax.fori_loop(0, bodies, body, 0)
        plsc.subcore_barrier()
        for k in range(ROWS_PER_TILE // ZR):
            rr = r0 + k * ZR
            pltpu.sync_copy(acc.at[pl.ds(rr, ZR)], out_hbm.at[cid, pl.ds(rr, ZR)])

        @pl.when(sid == NS - 1)
        def _write_tail():
            t0 = NS * ROWS_PER_TILE
            tn = NN - t0
            pltpu.sync_copy(acc.at[pl.ds(t0, tn)], out_hbm.at[cid, pl.ds(t0, tn)])

    return spmm


def kernel(adj_index, adj_values, image_adj_index, image_adj_values,
           text_adj_index, text_adj_values, image_embedding, text_embedding,
           u_embs, i_embs, W_img, b_img, W_txt, b_txt, modal_weight):
    ai = adj_index.astype(jnp.int32)
    ii = image_adj_index.astype(jnp.int32)
    ti = text_adj_index.astype(jnp.int32)
    _, pad3 = _padded_len(3 * EDGES)
    _, pad1 = _padded_len(EDGES)
    zi3 = jnp.zeros((pad3 - 3 * EDGES,), jnp.int32)
    zi1 = jnp.zeros((pad1 - EDGES,), jnp.int32)
    rows_all = jnp.concatenate([ai[0], ii[0], ti[0], zi3])
    cols_all = jnp.concatenate([ai[1], ii[1] + NN, ti[1] + 2 * NN, zi3])

    table, vals2d = _feats(
        image_embedding, W_img, b_img.reshape(1, D),
        text_embedding, W_txt, b_txt.reshape(1, D),
        u_embs, i_embs, modal_weight.reshape(1, 2),
        adj_values.reshape(-1, D), image_adj_values.reshape(-1, D),
        text_adj_values.reshape(-1, D),
    )
    vals_all = jnp.concatenate(
        [vals2d.reshape(-1), jnp.zeros((pad3 - 3 * EDGES,), jnp.float32)])

    p = _make_spmm(3 * EDGES)(rows_all, cols_all, vals_all, table)
    modal = _sum2(p)
    q = _make_spmm(EDGES)(
        jnp.concatenate([ai[0], zi1]), jnp.concatenate([ai[1], zi1]),
        jnp.concatenate([adj_values, zi1.astype(jnp.float32)]), modal)
    return _final(modal, q)


# speedup vs baseline: 1.7069x; 1.7069x over previous
import functools

import jax
import jax.numpy as jnp
from jax import lax
from jax.experimental import pallas as pl
from jax.experimental.pallas import tpu as pltpu
from jax.experimental.pallas import tpu_sc as plsc

USER = 6000
ITEM = 4000
NN = USER + ITEM
EDGES = 320000
D = 128
MODAL_ADJ_WEIGHT = 0.4
RESIDUAL_WEIGHT = 0.5

CH = 128
KB = 4
ZR = 208
NC = 2
NS = 16
NW = NC * NS
ROWS_PER_TILE = 624


def _l2n(x):
    n = jnp.sqrt(jnp.sum(x * x, axis=-1, keepdims=True))
    return x / jnp.maximum(n, 1e-12)


def _feats_body(img_ref, wimg_ref, bimg_ref, txt_ref, wtxt_ref, btxt_ref,
                u_ref, i_ref, mw_ref, vadj_ref, vimg_ref, vtxt_ref,
                t_ref, vals_ref):
    img_f = jnp.dot(img_ref[...], wimg_ref[...],
                    preferred_element_type=jnp.float32) + bimg_ref[...]
    txt_f = jnp.dot(txt_ref[...], wtxt_ref[...],
                    preferred_element_type=jnp.float32) + btxt_ref[...]
    img_n = _l2n(img_f)
    txt_n = _l2n(txt_f)
    u = u_ref[...]
    mw = mw_ref[...]
    m = jnp.exp(mw - jnp.max(mw))
    w = m / jnp.sum(m)
    w0 = w[0, 0]
    w1 = w[0, 1]
    t_ref[0:USER] = u
    t_ref[USER:NN] = i_ref[...]
    t_ref[NN:NN + USER] = u
    t_ref[NN + USER:2 * NN] = img_n
    t_ref[2 * NN:2 * NN + USER] = u
    t_ref[2 * NN + USER:3 * NN] = txt_n
    nb = EDGES // D
    vals_ref[0:nb] = vadj_ref[...]
    vals_ref[nb:2 * nb] = vimg_ref[...] * (MODAL_ADJ_WEIGHT * w0)
    vals_ref[2 * nb:3 * nb] = vtxt_ref[...] * (MODAL_ADJ_WEIGHT * w1)


_feats = pl.pallas_call(
    _feats_body,
    out_shape=(
        jax.ShapeDtypeStruct((3 * NN, D), jnp.float32),
        jax.ShapeDtypeStruct((3 * EDGES // D, D), jnp.float32),
    ),
)


def _sum2_body(p_ref, o_ref):
    o_ref[...] = p_ref[0] + p_ref[1]


_sum2 = pl.pallas_call(
    _sum2_body,
    out_shape=jax.ShapeDtypeStruct((NN, D), jnp.float32),
)


def _final_body(modal_ref, q_ref, o_ref):
    o_ref[...] = (1.0 + RESIDUAL_WEIGHT) * modal_ref[...] + q_ref[0] + q_ref[1]


_final = pl.pallas_call(
    _final_body,
    out_shape=jax.ShapeDtypeStruct((NN, D), jnp.float32),
)


def _padded_len(n_edges):
    blk = KB * NW * CH
    data = -(-n_edges // blk) * blk
    return data, data


@functools.cache
def _make_spmm(n_edges):
    data_len, _ = _padded_len(n_edges)
    cpw = data_len // (NW * CH)
    bodies = cpw // KB
    mesh = plsc.VectorSubcoreMesh(core_axis_name="c", subcore_axis_name="s")

    @functools.partial(
        pl.kernel,
        out_type=jax.ShapeDtypeStruct((NC, NN, D), jnp.float32),
        mesh=mesh,
        compiler_params=pltpu.CompilerParams(needs_layout_passes=False),
        scratch_types=[
            pltpu.VMEM((KB * CH,), jnp.int32),
            pltpu.VMEM((KB * CH,), jnp.int32),
            pltpu.VMEM((KB * CH,), jnp.float32),
            pltpu.VMEM((CH, D), jnp.float32),
            pltpu.VMEM((CH, D), jnp.float32),
            pltpu.VMEM((CH, D), jnp.float32),
            pltpu.VMEM_SHARED((NN, D), jnp.float32),
            pltpu.SemaphoreType.DMA,
            pltpu.SemaphoreType.DMA,
            pltpu.SemaphoreType.DMA,
        ],
    )
    def spmm(row_hbm, col_hbm, val_hbm, x_hbm, out_hbm,
             colv, rowv, valv, rb_0, rb_1, st32, acc, isem, gsem, ssem):
        cid = lax.axis_index("c")
        sid = lax.axis_index("s")
        wid = sid * NC + cid
        rbs = (rb_0, rb_1)

        def zero_st(i, carry):
            for g in range(D // 16):
                st32[i, pl.ds(g * 16, 16)] = jnp.zeros((16,), jnp.float32)
            return carry

        lax.fori_loop(0, CH, zero_st, 0)
        r0 = sid * ROWS_PER_TILE
        for k in range(ROWS_PER_TILE // CH):
            pltpu.sync_copy(st32, acc.at[pl.ds(r0 + k * CH, CH)])
        rem = ROWS_PER_TILE % CH
        pltpu.sync_copy(st32.at[pl.ds(0, rem)],
                        acc.at[pl.ds(r0 + ROWS_PER_TILE - rem, rem)])

        @pl.when(sid == NS - 1)
        def _zero_tail():
            pltpu.sync_copy(st32.at[pl.ds(0, NN - NS * ROWS_PER_TILE)],
                            acc.at[pl.ds(NS * ROWS_PER_TILE,
                                         NN - NS * ROWS_PER_TILE)])

        plsc.subcore_barrier()

        def scale(rbi, off):
            @plsc.parallel_loop(0, CH // 16)
            def _(i):
                vv = valv[pl.ds(off + i * 16, 16)]
                for j in range(16):
                    vj = jnp.broadcast_to(vv[j], (16,))
                    e = i * 16 + j
                    for g in range(D // 16):
                        sl = pl.ds(g * 16, 16)
                        st32[e, sl] = rbi[e, sl] * vj

        span0 = wid * cpw * CH

        def body(t, carry):
            base = span0 + t * (KB * CH)
            ic = pltpu.async_copy(col_hbm.at[pl.ds(base, KB * CH)], colv, isem)
            ir = pltpu.async_copy(row_hbm.at[pl.ds(base, KB * CH)], rowv, isem)
            iv = pltpu.async_copy(val_hbm.at[pl.ds(base, KB * CH)], valv, isem)
            HC = CH // 2

            def gather2(q):
                return (
                    pltpu.async_copy(
                        x_hbm.at[colv.at[pl.ds(q * CH, HC)]],
                        rbs[q % 2].at[pl.ds(0, HC)], gsem),
                    pltpu.async_copy(
                        x_hbm.at[colv.at[pl.ds(q * CH + HC, HC)]],
                        rbs[q % 2].at[pl.ds(HC, HC)], gsem),
                )

            ic.wait()
            g = [gather2(0), gather2(1)]
            ir.wait()
            iv.wait()
            s_prev = None
            for q in range(KB):
                g[q][0].wait()
                g[q][1].wait()
                if s_prev is not None:
                    s_prev.wait()
                scale(rbs[q % 2], q * CH)
                if q + 2 < KB:
                    g.append(gather2(q + 2))
                s_prev = pltpu.async_copy(
                    st32, acc.at[rowv.at[pl.ds(q * CH, CH)]], ssem, add=True)
            s_prev.wait()
            return carry

        lax.fori_loop(0, bodies, body, 0)
        plsc.subcore_barrier()
        for k in range(ROWS_PER_TILE // ZR):
            rr = r0 + k * ZR
            pltpu.sync_copy(acc.at[pl.ds(rr, ZR)], out_hbm.at[cid, pl.ds(rr, ZR)])

        @pl.when(sid == NS - 1)
        def _write_tail():
            t0 = NS * ROWS_PER_TILE
            tn = NN - t0
            pltpu.sync_copy(acc.at[pl.ds(t0, tn)], out_hbm.at[cid, pl.ds(t0, tn)])

    return spmm


def kernel(adj_index, adj_values, image_adj_index, image_adj_values,
           text_adj_index, text_adj_values, image_embedding, text_embedding,
           u_embs, i_embs, W_img, b_img, W_txt, b_txt, modal_weight):
    ai = adj_index.astype(jnp.int32)
    ii = image_adj_index.astype(jnp.int32)
    ti = text_adj_index.astype(jnp.int32)
    _, pad3 = _padded_len(3 * EDGES)
    _, pad1 = _padded_len(EDGES)
    zi3 = jnp.zeros((pad3 - 3 * EDGES,), jnp.int32)
    zi1 = jnp.zeros((pad1 - EDGES,), jnp.int32)
    rows_all = jnp.concatenate([ai[0], ii[0], ti[0], zi3])
    cols_all = jnp.concatenate([ai[1], ii[1] + NN, ti[1] + 2 * NN, zi3])

    table, vals2d = _feats(
        image_embedding, W_img, b_img.reshape(1, D),
        text_embedding, W_txt, b_txt.reshape(1, D),
        u_embs, i_embs, modal_weight.reshape(1, 2),
        adj_values.reshape(-1, D), image_adj_values.reshape(-1, D),
        text_adj_values.reshape(-1, D),
    )
    vals_all = jnp.concatenate(
        [vals2d.reshape(-1), jnp.zeros((pad3 - 3 * EDGES,), jnp.float32)])

    p = _make_spmm(3 * EDGES)(rows_all, cols_all, vals_all, table)
    modal = _sum2(p)
    q = _make_spmm(EDGES)(
        jnp.concatenate([ai[0], zi1]), jnp.concatenate([ai[1], zi1]),
        jnp.concatenate([adj_values, zi1.astype(jnp.float32)]), modal)
    return _final(modal, q)
